# baseline (speedup 1.0000x reference)
import functools

import jax
import jax.numpy as jnp
from jax.experimental import pallas as pl
from jax.experimental.pallas import tpu as pltpu

LANE = 128
SUBLANE = 8
TILE_B = 8192
VMEM_LIMIT_BYTES = 100 * 1024 * 1024


def _round_up(x, m):
    return (x + m - 1) // m * m


def _mlp_kernel(x_ref, w1_hbm, b1_hbm, w2_hbm, b2_hbm, o_ref,
                w1_s, b1_s, w2_s, b2_s, sems):
    j = pl.program_id(1)

    @pl.when(j == 0)
    def _load_params():
        pltpu.make_async_copy(w1_hbm, w1_s, sems.at[0]).start()
        pltpu.make_async_copy(b1_hbm, b1_s, sems.at[1]).start()
        pltpu.make_async_copy(w2_hbm, w2_s, sems.at[2]).start()
        pltpu.make_async_copy(b2_hbm, b2_s, sems.at[3]).start()
        pltpu.make_async_copy(w1_hbm, w1_s, sems.at[0]).wait()
        pltpu.make_async_copy(b1_hbm, b1_s, sems.at[1]).wait()
        pltpu.make_async_copy(w2_hbm, w2_s, sems.at[2]).wait()
        pltpu.make_async_copy(b2_hbm, b2_s, sems.at[3]).wait()

    x = x_ref[...].astype(jnp.bfloat16)
    w1 = w1_s[...].astype(jnp.bfloat16)
    h = jnp.dot(x, w1, preferred_element_type=jnp.float32)
    b1b = b1_s[...].astype(jnp.bfloat16)
    h = jnp.maximum(h.astype(jnp.bfloat16) + b1b, jnp.bfloat16(0.0))
    w2 = w2_s[...].astype(jnp.bfloat16)
    out = jnp.dot(h, w2, preferred_element_type=jnp.float32)
    o_ref[...] = out + b2_s[...]


def kernel(x, w1, b1, w2p, b2p):
    B, D = x.shape
    H = w1.shape[1]
    A = w2p.shape[1]
    A_pad = max(_round_up(A, LANE), LANE)
    if A_pad != A:
        w2p = jnp.pad(w2p, ((0, 0), (0, A_pad - A)))
        b2p = jnp.pad(b2p, ((0, 0), (0, A_pad - A)))

    tb = min(TILE_B, _round_up(B, SUBLANE))
    B_pad = _round_up(B, tb)
    if B_pad != B:
        x = jnp.pad(x, ((0, B_pad - B), (0, 0)))
    n_tiles = B_pad // tb
    if n_tiles % 2 == 0:
        n_cores, n_inner = 2, n_tiles // 2
    else:
        n_cores, n_inner = 1, n_tiles

    vmem = pltpu.MemorySpace.VMEM
    hbm = pltpu.MemorySpace.HBM

    out = pl.pallas_call(
        _mlp_kernel,
        out_shape=jax.ShapeDtypeStruct((B_pad, A_pad), jnp.float32),
        grid=(n_cores, n_inner),
        in_specs=[
            pl.BlockSpec((tb, D),
                         functools.partial(
                             lambda n, c, j: (c * n + j, 0), n_inner)),
            pl.BlockSpec(memory_space=hbm),
            pl.BlockSpec(memory_space=hbm),
            pl.BlockSpec(memory_space=hbm),
            pl.BlockSpec(memory_space=hbm),
        ],
        out_specs=pl.BlockSpec((tb, A_pad),
                               functools.partial(
                                   lambda n, c, j: (c * n + j, 0), n_inner)),
        scratch_shapes=[
            pltpu.VMEM((D, H), jnp.float32),
            pltpu.VMEM((1, H), jnp.float32),
            pltpu.VMEM((H, A_pad), jnp.float32),
            pltpu.VMEM((1, A_pad), jnp.float32),
            pltpu.SemaphoreType.DMA((4,)),
        ],
        compiler_params=pltpu.CompilerParams(
            dimension_semantics=("parallel", "arbitrary"),
            vmem_limit_bytes=VMEM_LIMIT_BYTES,
        ),
    )(x, w1, b1, w2p, b2p)

    return out[:B, :A]


# speedup vs baseline: 1.1526x; 1.1526x over previous
import jax
import jax.numpy as jnp
from jax.experimental import pallas as pl
from jax.experimental.pallas import tpu as pltpu

LANE = 128
SUBLANE = 8
TILE_B = 8192
VMEM_LIMIT_BYTES = 100 * 1024 * 1024


def _round_up(x, m):
    return (x + m - 1) // m * m


def _mlp_kernel(x_ref, w1_ref, b1_ref, w2_ref, b2_ref, o_ref):
    h = jnp.dot(x_ref[...], w1_ref[...], preferred_element_type=jnp.float32)
    h = jnp.maximum(h + b1_ref[...], 0.0)
    out = jnp.dot(h, w2_ref[...], preferred_element_type=jnp.float32)
    o_ref[...] = out + b2_ref[...]


def kernel(x, w1, b1, w2p, b2p):
    B, D = x.shape
    H = w1.shape[1]
    A = w2p.shape[1]
    A_pad = max(_round_up(A, LANE), LANE)
    if A_pad != A:
        w2p = jnp.pad(w2p, ((0, 0), (0, A_pad - A)))
        b2p = jnp.pad(b2p, ((0, 0), (0, A_pad - A)))

    tb = min(TILE_B, _round_up(B, SUBLANE))
    B_pad = _round_up(B, tb)
    if B_pad != B:
        x = jnp.pad(x, ((0, B_pad - B), (0, 0)))
    n_tiles = B_pad // tb

    out = pl.pallas_call(
        _mlp_kernel,
        out_shape=jax.ShapeDtypeStruct((B_pad, A_pad), jnp.float32),
        grid=(n_tiles,),
        in_specs=[
            pl.BlockSpec((tb, D), lambda i: (i, 0)),
            pl.BlockSpec((D, H), lambda i: (0, 0)),
            pl.BlockSpec((1, H), lambda i: (0, 0)),
            pl.BlockSpec((H, A_pad), lambda i: (0, 0)),
            pl.BlockSpec((1, A_pad), lambda i: (0, 0)),
        ],
        out_specs=pl.BlockSpec((tb, A_pad), lambda i: (i, 0)),
        compiler_params=pltpu.CompilerParams(
            dimension_semantics=("parallel",),
            vmem_limit_bytes=VMEM_LIMIT_BYTES,
        ),
    )(x, w1, b1, w2p, b2p)

    return out[:B, :A]
